# baseline (speedup 1.0000x reference)
import functools

import jax
import jax.numpy as jnp
from jax import lax
from jax.experimental import pallas as pl
from jax.experimental.pallas import tpu as pltpu
from jax.experimental.pallas import tpu_sc as plsc

N_NODES = 10000
C = 128
N_PAD = 10240
NC = 2
NS = 16
NT = NC * NS
CHUNK = 128
CPT = 80
E_PAD = NT * CPT * CHUNK
RPS = N_PAD // NS
BR = 512

_mesh = plsc.VectorSubcoreMesh(
    core_axis_name="c", subcore_axis_name="s", num_cores=NC, num_subcores=NS)


@functools.partial(
    pl.kernel,
    out_type=jax.ShapeDtypeStruct((NC, N_PAD), jnp.float32),
    mesh=_mesh,
    scratch_types=[
        pltpu.VMEM((CPT, CHUNK), jnp.int32),
        pltpu.VMEM((CHUNK,), jnp.float32),
        pltpu.VMEM_SHARED((N_PAD,), jnp.float32),
    ],
)
def _deg_kernel(dst_hbm, ones_hbm, zeros_hbm, deg_out, idx_v, ones_v, deg_sh):
    cid = lax.axis_index("c")
    sid = lax.axis_index("s")
    wid = sid * NC + cid
    pltpu.sync_copy(zeros_hbm, deg_sh.at[pl.ds(sid * RPS, RPS)])
    pltpu.sync_copy(ones_hbm, ones_v)
    pltpu.sync_copy(dst_hbm.at[wid], idx_v)
    plsc.subcore_barrier()

    @pl.loop(0, CPT)
    def _(j):
        pltpu.sync_copy(ones_v, deg_sh.at[idx_v.at[j]], add=True)

    plsc.subcore_barrier()
    pltpu.sync_copy(deg_sh.at[pl.ds(sid * RPS, RPS)],
                    deg_out.at[cid, pl.ds(sid * RPS, RPS)])


NBUF = 2
HCPT = CPT // 2


@functools.partial(
    pl.kernel,
    out_type=jax.ShapeDtypeStruct((NC, N_PAD, C), jnp.float32),
    mesh=_mesh,
    scratch_types=(
        [pltpu.VMEM((HCPT, CHUNK), jnp.int32)] * 2 +
        [pltpu.VMEM((CHUNK, C), jnp.float32)] * NBUF +
        [pltpu.VMEM_SHARED((N_PAD, C), jnp.float32)] +
        [pltpu.SemaphoreType.DMA] * (2 * NBUF + 1)
    ),
)
def _agg_kernel(g_hbm, src_hbm, dstr_hbm, zeros_hbm, acc_out,
                idxs_v, idxd_v, b0, b1, acc_sh, *sems):
    bufs = (b0, b1)
    gsems = sems[:NBUF]
    ssems = sems[NBUF:2 * NBUF]
    misc = sems[2 * NBUF]
    cid = lax.axis_index("c")
    sid = lax.axis_index("s")
    wid = sid * NC + cid

    c_z = pltpu.async_copy(zeros_hbm, b0, misc)
    c_z.wait()
    for r in range(RPS // CHUNK):
        pltpu.async_copy(b0, acc_sh.at[pl.ds(sid * RPS + r * CHUNK, CHUNK)],
                         misc)
    for r in range(RPS // CHUNK):
        pltpu.make_async_copy(b0, acc_sh.at[pl.ds(0, CHUNK)], misc).wait()
    plsc.subcore_barrier()

    def g_copy(j, bi):
        return g_hbm.at[idxs_v.at[j]], bufs[bi], gsems[bi]

    def s_copy(j, bi):
        return bufs[bi], acc_sh.at[idxd_v.at[j]], ssems[bi]

    for half in range(2):
        base = half * HCPT
        c_src = pltpu.async_copy(src_hbm.at[wid, pl.ds(base, HCPT)],
                                 idxs_v, gsems[0])
        c_dst = pltpu.async_copy(dstr_hbm.at[wid, pl.ds(base, HCPT)],
                                 idxd_v, gsems[1])
        c_src.wait()
        c_dst.wait()
        for bi in range(NBUF):
            pltpu.async_copy(*g_copy(bi, bi))

        @pl.loop(0, HCPT - NBUF, step=NBUF)
        def _(j0):
            for bi in range(NBUF):
                j = j0 + bi
                pltpu.make_async_copy(*g_copy(j, bi)).wait()
                pltpu.async_copy(*s_copy(j, bi), add=True)
                pltpu.make_async_copy(*s_copy(j, bi)).wait()
                pltpu.async_copy(*g_copy(j + NBUF, bi))

        for bi in range(NBUF):
            j = HCPT - NBUF + bi
            pltpu.make_async_copy(*g_copy(j, bi)).wait()
            pltpu.async_copy(*s_copy(j, bi), add=True)
        for bi in range(NBUF):
            pltpu.make_async_copy(*s_copy(HCPT - NBUF + bi, bi)).wait()

    plsc.subcore_barrier()
    for r in range(RPS // CHUNK):
        row = sid * RPS + r * CHUNK
        pltpu.async_copy(acc_sh.at[pl.ds(row, CHUNK)],
                         acc_out.at[cid, pl.ds(row, CHUNK)], misc)
    for r in range(RPS // CHUNK):
        row = sid * RPS + r * CHUNK
        pltpu.make_async_copy(acc_sh.at[pl.ds(row, CHUNK)],
                              acc_out.at[cid, pl.ds(row, CHUNK)], misc).wait()


def _mm_body(x_ref, w_ref, o_ref):
    o_ref[...] = jnp.dot(x_ref[...], w_ref[...],
                         preferred_element_type=jnp.float32)


def _matmul(x_p, W):
    return pl.pallas_call(
        _mm_body,
        grid=(N_PAD // BR,),
        in_specs=[
            pl.BlockSpec((BR, C), lambda i: (i, 0)),
            pl.BlockSpec((C, C), lambda i: (0, 0)),
        ],
        out_specs=pl.BlockSpec((BR, C), lambda i: (i, 0)),
        out_shape=jax.ShapeDtypeStruct((N_PAD, C), jnp.float32),
    )(x_p, W)


def _scale_body(h_ref, d_ref, g_ref):
    deg = d_ref[:, 0:1] + d_ref[:, 1:2] + 1.0
    g_ref[...] = lax.rsqrt(deg) * h_ref[...]


def _scale(h, deg_t):
    return pl.pallas_call(
        _scale_body,
        grid=(N_PAD // BR,),
        in_specs=[
            pl.BlockSpec((BR, C), lambda i: (i, 0)),
            pl.BlockSpec((BR, NC), lambda i: (i, 0)),
        ],
        out_specs=pl.BlockSpec((BR, C), lambda i: (i, 0)),
        out_shape=jax.ShapeDtypeStruct((N_PAD, C), jnp.float32),
    )(h, deg_t)


def _ep_body(a_ref, g_ref, d_ref, b_ref, o_ref):
    deg = d_ref[:, 0:1] + d_ref[:, 1:2] + 1.0
    acc = a_ref[0] + a_ref[1] + g_ref[...]
    o_ref[...] = lax.rsqrt(deg) * acc + b_ref[...]


def _epilogue(acc_parts, g, deg_t, b2d):
    return pl.pallas_call(
        _ep_body,
        grid=(N_PAD // BR,),
        in_specs=[
            pl.BlockSpec((NC, BR, C), lambda i: (0, i, 0)),
            pl.BlockSpec((BR, C), lambda i: (i, 0)),
            pl.BlockSpec((BR, NC), lambda i: (i, 0)),
            pl.BlockSpec((1, C), lambda i: (0, 0)),
        ],
        out_specs=pl.BlockSpec((BR, C), lambda i: (i, 0)),
        out_shape=jax.ShapeDtypeStruct((N_PAD, C), jnp.float32),
    )(acc_parts, g, deg_t, b2d)


def kernel(x, adj_t, W, b):
    src = adj_t[0].astype(jnp.int32)
    dst = adj_t[1].astype(jnp.int32)
    n_edges = src.shape[0]
    pad = E_PAD - n_edges
    junk = N_NODES + jnp.arange(pad, dtype=jnp.int32) % (N_PAD - N_NODES)
    src_p = jnp.concatenate([src, jnp.zeros((pad,), jnp.int32)])
    dst_p = jnp.concatenate([dst, junk])
    src_r = src_p.reshape(NT, CPT, CHUNK)
    dst_r = dst_p.reshape(NT, CPT, CHUNK)
    x_p = jnp.pad(x, ((0, N_PAD - x.shape[0]), (0, 0)))

    ones_c = jnp.ones((CHUNK,), jnp.float32)
    zeros_r = jnp.zeros((RPS,), jnp.float32)
    zeros_c = jnp.zeros((CHUNK, C), jnp.float32)

    deg_parts = _deg_kernel(dst_r, ones_c, zeros_r)
    deg_t = deg_parts.T
    h = _matmul(x_p, W)
    g = _scale(h, deg_t)
    acc_parts = _agg_kernel(g, src_r, dst_r, zeros_c)
    out = _epilogue(acc_parts, g, deg_t, b.reshape(1, C))
    return out[:N_NODES]


# speedup vs baseline: 2.5587x; 2.5587x over previous
import functools

import jax
import jax.numpy as jnp
from jax import lax
from jax.experimental import pallas as pl
from jax.experimental.pallas import tpu as pltpu
from jax.experimental.pallas import tpu_sc as plsc

N_NODES = 10000
C = 128
N_PAD = 10240
NC = 2
NS = 16
NT = NC * NS
CHUNK = 128
CPT = 80
E_PAD = NT * CPT * CHUNK
RPS = N_PAD // NS
BR = 512

_mesh = plsc.VectorSubcoreMesh(
    core_axis_name="c", subcore_axis_name="s", num_cores=NC, num_subcores=NS)


@functools.partial(
    pl.kernel,
    out_type=jax.ShapeDtypeStruct((NC, N_PAD), jnp.float32),
    mesh=_mesh,
    scratch_types=[
        pltpu.VMEM((CPT, CHUNK), jnp.int32),
        pltpu.VMEM((CHUNK,), jnp.float32),
        pltpu.VMEM_SHARED((N_PAD,), jnp.float32),
    ],
)
def _deg_kernel(dst_hbm, ones_hbm, zeros_hbm, deg_out, idx_v, ones_v, deg_sh):
    cid = lax.axis_index("c")
    sid = lax.axis_index("s")
    wid = sid * NC + cid
    pltpu.sync_copy(zeros_hbm, deg_sh.at[pl.ds(sid * RPS, RPS)])
    pltpu.sync_copy(ones_hbm, ones_v)
    pltpu.sync_copy(dst_hbm.at[wid], idx_v)
    plsc.subcore_barrier()

    @pl.loop(0, CPT)
    def _(j):
        pltpu.sync_copy(ones_v, deg_sh.at[idx_v.at[j]], add=True)

    plsc.subcore_barrier()
    pltpu.sync_copy(deg_sh.at[pl.ds(sid * RPS, RPS)],
                    deg_out.at[cid, pl.ds(sid * RPS, RPS)])


NBUF = 2
HCPT = CPT // 2


@functools.partial(
    pl.kernel,
    out_type=jax.ShapeDtypeStruct((NC, N_PAD, C), jnp.float32),
    mesh=_mesh,
    scratch_types=(
        [pltpu.VMEM((HCPT, CHUNK), jnp.int32)] * 2 +
        [pltpu.VMEM((CHUNK, C), jnp.float32)] * NBUF +
        [pltpu.VMEM_SHARED((N_PAD, C), jnp.float32)] +
        [pltpu.SemaphoreType.DMA] * (2 * NBUF + 1)
    ),
)
def _agg_kernel(g_hbm, src_hbm, dstr_hbm, zeros_hbm, acc_out,
                idxs_v, idxd_v, b0, b1, acc_sh, *sems):
    bufs = (b0, b1)
    gsems = sems[:NBUF]
    ssems = sems[NBUF:2 * NBUF]
    misc = sems[2 * NBUF]
    cid = lax.axis_index("c")
    sid = lax.axis_index("s")
    wid = sid * NC + cid

    c_z = pltpu.async_copy(zeros_hbm, b0, misc)
    c_z.wait()
    for r in range(RPS // CHUNK):
        pltpu.async_copy(b0, acc_sh.at[pl.ds(sid * RPS + r * CHUNK, CHUNK)],
                         misc)
    for r in range(RPS // CHUNK):
        pltpu.make_async_copy(b0, acc_sh.at[pl.ds(0, CHUNK)], misc).wait()
    plsc.subcore_barrier()

    def g_copy(j, bi):
        return g_hbm.at[idxs_v.at[j]], bufs[bi], gsems[bi]

    def s_copy(j, bi):
        return bufs[bi], acc_sh.at[idxd_v.at[j]], ssems[bi]

    for half in range(2):
        base = half * HCPT
        c_src = pltpu.async_copy(src_hbm.at[wid, pl.ds(base, HCPT)],
                                 idxs_v, gsems[0])
        c_dst = pltpu.async_copy(dstr_hbm.at[wid, pl.ds(base, HCPT)],
                                 idxd_v, gsems[1])
        c_src.wait()
        c_dst.wait()
        for bi in range(NBUF):
            pltpu.async_copy(*g_copy(bi, bi))

        @pl.loop(0, HCPT - NBUF, step=NBUF)
        def _(j0):
            for bi in range(NBUF):
                j = j0 + bi
                pltpu.make_async_copy(*g_copy(j, bi)).wait()
                pltpu.async_copy(*s_copy(j, bi), add=True)
                pltpu.make_async_copy(*s_copy(j, bi)).wait()
                pltpu.async_copy(*g_copy(j + NBUF, bi))

        for bi in range(NBUF):
            j = HCPT - NBUF + bi
            pltpu.make_async_copy(*g_copy(j, bi)).wait()
            pltpu.async_copy(*s_copy(j, bi), add=True)
        for bi in range(NBUF):
            pltpu.make_async_copy(*s_copy(HCPT - NBUF + bi, bi)).wait()

    plsc.subcore_barrier()
    for r in range(RPS // CHUNK):
        row = sid * RPS + r * CHUNK
        pltpu.async_copy(acc_sh.at[pl.ds(row, CHUNK)],
                         acc_out.at[cid, pl.ds(row, CHUNK)], misc)
    for r in range(RPS // CHUNK):
        row = sid * RPS + r * CHUNK
        pltpu.make_async_copy(acc_sh.at[pl.ds(row, CHUNK)],
                              acc_out.at[cid, pl.ds(row, CHUNK)], misc).wait()


def _mm_body(x_ref, w_ref, d_ref, o_ref):
    deg = d_ref[:, 0:1] + d_ref[:, 1:2] + 1.0
    o_ref[...] = lax.rsqrt(deg) * jnp.dot(x_ref[...], w_ref[...],
                                          preferred_element_type=jnp.float32)


def _matmul_scale(x_p, W, deg_t):
    return pl.pallas_call(
        _mm_body,
        grid=(N_PAD // BR,),
        in_specs=[
            pl.BlockSpec((BR, C), lambda i: (i, 0)),
            pl.BlockSpec((C, C), lambda i: (0, 0)),
            pl.BlockSpec((BR, NC), lambda i: (i, 0)),
        ],
        out_specs=pl.BlockSpec((BR, C), lambda i: (i, 0)),
        out_shape=jax.ShapeDtypeStruct((N_PAD, C), jnp.float32),
    )(x_p, W, deg_t)


def _ep_body(a_ref, g_ref, d_ref, b_ref, o_ref):
    deg = d_ref[:, 0:1] + d_ref[:, 1:2] + 1.0
    acc = a_ref[0] + a_ref[1] + g_ref[...]
    o_ref[...] = lax.rsqrt(deg) * acc + b_ref[...]


def _epilogue(acc_parts, g, deg_t, b2d):
    return pl.pallas_call(
        _ep_body,
        grid=(N_PAD // BR,),
        in_specs=[
            pl.BlockSpec((NC, BR, C), lambda i: (0, i, 0)),
            pl.BlockSpec((BR, C), lambda i: (i, 0)),
            pl.BlockSpec((BR, NC), lambda i: (i, 0)),
            pl.BlockSpec((1, C), lambda i: (0, 0)),
        ],
        out_specs=pl.BlockSpec((BR, C), lambda i: (i, 0)),
        out_shape=jax.ShapeDtypeStruct((N_PAD, C), jnp.float32),
    )(acc_parts, g, deg_t, b2d)


def kernel(x, adj_t, W, b):
    src = adj_t[0].astype(jnp.int32)
    dst = adj_t[1].astype(jnp.int32)
    n_edges = src.shape[0]
    pad = E_PAD - n_edges
    junk = N_NODES + jnp.arange(pad, dtype=jnp.int32) % (N_PAD - N_NODES)
    src_p = jnp.concatenate([src, junk])
    dst_p = jnp.concatenate([dst, junk])
    src_r = src_p.reshape(NT, CPT, CHUNK)
    dst_r = dst_p.reshape(NT, CPT, CHUNK)
    x_p = jnp.pad(x, ((0, N_PAD - x.shape[0]), (0, 0)))

    ones_c = jnp.ones((CHUNK,), jnp.float32)
    zeros_r = jnp.zeros((RPS,), jnp.float32)
    zeros_c = jnp.zeros((CHUNK, C), jnp.float32)

    deg_parts = _deg_kernel(dst_r, ones_c, zeros_r)
    deg_t = deg_parts.T
    g = _matmul_scale(x_p, W, deg_t)
    acc_parts = _agg_kernel(g, src_r, dst_r, zeros_c)
    out = _epilogue(acc_parts, g, deg_t, b.reshape(1, C))
    return out[:N_NODES]


# speedup vs baseline: 2.6080x; 1.0193x over previous
import functools

import numpy as np

import jax
import jax.numpy as jnp
from jax import lax
from jax.experimental import pallas as pl
from jax.experimental.pallas import tpu as pltpu
from jax.experimental.pallas import tpu_sc as plsc

N_NODES = 10000
C = 128
N_PAD = 10240
NC = 2
NS = 16
NT = NC * NS
CHUNK = 128
CPT = 80
E_PAD = NT * CPT * CHUNK
RPS = N_PAD // NS
BR = 512

_mesh = plsc.VectorSubcoreMesh(
    core_axis_name="c", subcore_axis_name="s", num_cores=NC, num_subcores=NS)


@functools.partial(
    pl.kernel,
    out_type=jax.ShapeDtypeStruct((NC, N_PAD), jnp.float32),
    mesh=_mesh,
    scratch_types=[
        pltpu.VMEM((CPT, CHUNK), jnp.int32),
        pltpu.VMEM((CHUNK,), jnp.float32),
        pltpu.VMEM_SHARED((N_PAD,), jnp.float32),
    ],
)
def _deg_kernel(dst_hbm, ones_hbm, zeros_hbm, deg_out, idx_v, ones_v, deg_sh):
    cid = lax.axis_index("c")
    sid = lax.axis_index("s")
    wid = sid * NC + cid
    pltpu.sync_copy(zeros_hbm, deg_sh.at[pl.ds(sid * RPS, RPS)])
    pltpu.sync_copy(ones_hbm, ones_v)
    pltpu.sync_copy(dst_hbm.at[wid], idx_v)
    plsc.subcore_barrier()

    @pl.loop(0, CPT)
    def _(j):
        pltpu.sync_copy(ones_v, deg_sh.at[idx_v.at[j]], add=True)

    plsc.subcore_barrier()
    pltpu.sync_copy(deg_sh.at[pl.ds(sid * RPS, RPS)],
                    deg_out.at[cid, pl.ds(sid * RPS, RPS)])


NBUF = 2
HCPT = CPT // 2


@functools.partial(
    pl.kernel,
    out_type=jax.ShapeDtypeStruct((NC, N_PAD, C), jnp.float32),
    mesh=_mesh,
    scratch_types=(
        [pltpu.VMEM((HCPT, CHUNK), jnp.int32)] * 2 +
        [pltpu.VMEM((CHUNK, C), jnp.float32)] * NBUF +
        [pltpu.VMEM_SHARED((N_PAD, C), jnp.float32)] +
        [pltpu.SemaphoreType.DMA] * (2 * NBUF + 1)
    ),
)
def _agg_kernel(g_hbm, src_hbm, dstr_hbm, zeros_hbm, acc_out,
                idxs_v, idxd_v, b0, b1, acc_sh, *sems):
    bufs = (b0, b1)
    gsems = sems[:NBUF]
    ssems = sems[NBUF:2 * NBUF]
    misc = sems[2 * NBUF]
    cid = lax.axis_index("c")
    sid = lax.axis_index("s")
    wid = sid * NC + cid

    c_z = pltpu.async_copy(zeros_hbm, b0, misc)
    c_z.wait()
    for r in range(RPS // CHUNK):
        pltpu.async_copy(b0, acc_sh.at[pl.ds(sid * RPS + r * CHUNK, CHUNK)],
                         misc)
    for r in range(RPS // CHUNK):
        pltpu.make_async_copy(b0, acc_sh.at[pl.ds(0, CHUNK)], misc).wait()
    plsc.subcore_barrier()

    def g_copy(j, bi):
        return g_hbm.at[idxs_v.at[j]], bufs[bi], gsems[bi]

    def s_copy(j, bi):
        return bufs[bi], acc_sh.at[idxd_v.at[j]], ssems[bi]

    for half in range(2):
        base = half * HCPT
        c_src = pltpu.async_copy(src_hbm.at[wid, pl.ds(base, HCPT)],
                                 idxs_v, gsems[0])
        c_dst = pltpu.async_copy(dstr_hbm.at[wid, pl.ds(base, HCPT)],
                                 idxd_v, gsems[1])
        c_src.wait()
        c_dst.wait()
        for bi in range(NBUF):
            pltpu.async_copy(*g_copy(bi, bi))

        @pl.loop(0, HCPT - NBUF, step=NBUF)
        def _(j0):
            for bi in range(NBUF):
                j = j0 + bi
                pltpu.make_async_copy(*g_copy(j, bi)).wait()
                pltpu.async_copy(*s_copy(j, bi), add=True)
                pltpu.make_async_copy(*s_copy(j, bi)).wait()
                pltpu.async_copy(*g_copy(j + NBUF, bi))

        for bi in range(NBUF):
            j = HCPT - NBUF + bi
            pltpu.make_async_copy(*g_copy(j, bi)).wait()
            pltpu.async_copy(*s_copy(j, bi), add=True)
        for bi in range(NBUF):
            pltpu.make_async_copy(*s_copy(HCPT - NBUF + bi, bi)).wait()

    plsc.subcore_barrier()
    for r in range(RPS // CHUNK):
        row = sid * RPS + r * CHUNK
        pltpu.async_copy(acc_sh.at[pl.ds(row, CHUNK)],
                         acc_out.at[cid, pl.ds(row, CHUNK)], misc)
    for r in range(RPS // CHUNK):
        row = sid * RPS + r * CHUNK
        pltpu.make_async_copy(acc_sh.at[pl.ds(row, CHUNK)],
                              acc_out.at[cid, pl.ds(row, CHUNK)], misc).wait()


def _mm_body(x_ref, w_ref, d_ref, o_ref):
    deg = d_ref[:, 0:1] + d_ref[:, 1:2] + 1.0
    o_ref[...] = lax.rsqrt(deg) * jnp.dot(x_ref[...], w_ref[...],
                                          preferred_element_type=jnp.float32)


def _matmul_scale(x_p, W, deg_t):
    return pl.pallas_call(
        _mm_body,
        grid=(N_PAD // BR,),
        in_specs=[
            pl.BlockSpec((BR, C), lambda i: (i, 0)),
            pl.BlockSpec((C, C), lambda i: (0, 0)),
            pl.BlockSpec((BR, NC), lambda i: (i, 0)),
        ],
        out_specs=pl.BlockSpec((BR, C), lambda i: (i, 0)),
        out_shape=jax.ShapeDtypeStruct((N_PAD, C), jnp.float32),
    )(x_p, W, deg_t)


def _ep_body(a_ref, g_ref, d_ref, b_ref, o_ref):
    deg = d_ref[:, 0:1] + d_ref[:, 1:2] + 1.0
    acc = a_ref[0] + a_ref[1] + g_ref[...]
    o_ref[...] = lax.rsqrt(deg) * acc + b_ref[...]


BRE = 400


def _epilogue(acc_parts, g, deg_t, b2d):
    return pl.pallas_call(
        _ep_body,
        grid=(N_NODES // BRE,),
        in_specs=[
            pl.BlockSpec((NC, BRE, C), lambda i: (0, i, 0)),
            pl.BlockSpec((BRE, C), lambda i: (i, 0)),
            pl.BlockSpec((BRE, NC), lambda i: (i, 0)),
            pl.BlockSpec((1, C), lambda i: (0, 0)),
        ],
        out_specs=pl.BlockSpec((BRE, C), lambda i: (i, 0)),
        out_shape=jax.ShapeDtypeStruct((N_NODES, C), jnp.float32),
    )(acc_parts, g, deg_t, b2d)


def kernel(x, adj_t, W, b):
    src = adj_t[0].astype(jnp.int32)
    dst = adj_t[1].astype(jnp.int32)
    n_edges = src.shape[0]
    pad = E_PAD - n_edges
    junk = jnp.asarray(
        N_NODES + np.arange(pad, dtype=np.int32) % (N_PAD - N_NODES))
    src_p = jnp.concatenate([src, junk])
    dst_p = jnp.concatenate([dst, junk])
    src_r = src_p.reshape(NT, CPT, CHUNK)
    dst_r = dst_p.reshape(NT, CPT, CHUNK)
    x_p = jnp.pad(x, ((0, N_PAD - x.shape[0]), (0, 0)))

    ones_c = jnp.ones((CHUNK,), jnp.float32)
    zeros_r = jnp.zeros((RPS,), jnp.float32)
    zeros_c = jnp.zeros((CHUNK, C), jnp.float32)

    deg_parts = _deg_kernel(dst_r, ones_c, zeros_r)
    deg_t = deg_parts.T
    g = _matmul_scale(x_p, W, deg_t)
    acc_parts = _agg_kernel(g, src_r, dst_r, zeros_c)
    return _epilogue(acc_parts, g, deg_t, b.reshape(1, C))
